# baseline (speedup 1.0000x reference)
import functools

import jax
import jax.numpy as jnp
from jax import lax
from jax.experimental import pallas as pl
from jax.experimental.pallas import tpu as pltpu
from jax.experimental.pallas import tpu_sc as plsc

NC = 2
NS = 16
L = 16
NW = NC * NS

N_NODES = 100_000
N_EDGES = 6_400_000
EW = N_EDGES // NW
B = 2_000
NCHUNK = EW // B
G = B // L
UNROLL = 5

_mesh = plsc.VectorSubcoreMesh(
    core_axis_name="c", subcore_axis_name="s", num_cores=NC, num_subcores=NS
)


@functools.partial(
    pl.kernel,
    out_type=jax.ShapeDtypeStruct((N_EDGES,), jnp.int32),
    mesh=_mesh,
    scratch_types=(
        [pltpu.VMEM_SHARED((4 * N_NODES,), jnp.float32)]
        + [pltpu.VMEM((8 * B,), jnp.int32) for _ in range(2)]
        + [pltpu.VMEM((8 * B,), jnp.float32) for _ in range(2)]
        + [pltpu.SemaphoreType.DMA for _ in range(2)]
    ),
)
def _ef_kernel(
    cat_hbm, ei_hbm, ej_hbm, out_hbm,
    cat_sh,
    c0, c1, f0, f1,
    sem0, sem1,
):
    wid = lax.axis_index("s") * NC + lax.axis_index("c")
    sid = lax.axis_index("s")

    @pl.when(sid == 0)
    def _():
        pltpu.sync_copy(cat_hbm, cat_sh)

    plsc.subcore_barrier()

    bufs = [
        dict(c=c0, f=f0, sem=sem0),
        dict(c=c1, f=f1, sem=sem1),
    ]

    def prep_and_fire(ch, bs):
        base = wid * EW + ch * B
        cv = bs["c"]
        pltpu.sync_copy(ei_hbm.at[pl.ds(base, B)], cv.at[pl.ds(0, B)])
        pltpu.sync_copy(ej_hbm.at[pl.ds(base, B)], cv.at[pl.ds(4 * B, B)])

        def build_body(g, carry):
            sl0 = g * L
            vi = cv[pl.ds(sl0, L)]
            vj = cv[pl.ds(4 * B + sl0, L)]
            cv[pl.ds(B + sl0, L)] = vi + N_NODES
            cv[pl.ds(2 * B + sl0, L)] = vi + 2 * N_NODES
            cv[pl.ds(3 * B + sl0, L)] = vi + 3 * N_NODES
            cv[pl.ds(5 * B + sl0, L)] = vj + N_NODES
            cv[pl.ds(6 * B + sl0, L)] = vj + 2 * N_NODES
            cv[pl.ds(7 * B + sl0, L)] = vj + 3 * N_NODES
            return carry

        lax.fori_loop(0, G, build_body, 0, unroll=4)
        pltpu.async_copy(cat_sh.at[cv], bs["f"], bs["sem"])

    def wait_gather(bs):
        pltpu.make_async_copy(cat_sh.at[bs["c"]], bs["f"], bs["sem"]).wait()

    def compute(ch, bs):
        fv = bs["f"]
        o_v = bs["c"]

        def group_body(g, gcarry):
            sl0 = g * L
            ri = fv[pl.ds(sl0, L)]
            phii = fv[pl.ds(B + sl0, L)]
            zi = fv[pl.ds(2 * B + sl0, L)]
            etai = fv[pl.ds(3 * B + sl0, L)]
            rj = fv[pl.ds(4 * B + sl0, L)]
            phij = fv[pl.ds(5 * B + sl0, L)]
            zj = fv[pl.ds(6 * B + sl0, L)]
            etaj = fv[pl.ds(7 * B + sl0, L)]
            dz = zi - zj
            dr = ri - rj
            dphi = phii - phij
            deta = etai - etaj
            s = deta * deta + dphi * dphi
            z0 = zi - ri * dz / dr
            m = (
                (dphi * dphi < 3.6e-05 * s)
                & (jnp.abs(z0) < 150.0)
                & (s < 2.89)
            )
            o_v[pl.ds(sl0, L)] = jnp.where(m, 1, 0).astype(jnp.int32)
            return gcarry

        lax.fori_loop(0, G, group_body, 0, unroll=UNROLL)
        base = wid * EW + ch * B
        pltpu.sync_copy(o_v.at[pl.ds(0, B)], out_hbm.at[pl.ds(base, B)])

    prep_and_fire(0, bufs[0])

    def pair_body(t, carry):
        ch = 2 * t
        prep_and_fire(ch + 1, bufs[1])
        wait_gather(bufs[0])
        compute(ch, bufs[0])

        @pl.when(t < NCHUNK // 2 - 1)
        def _():
            prep_and_fire(ch + 2, bufs[0])

        wait_gather(bufs[1])
        compute(ch + 1, bufs[1])
        return carry

    lax.fori_loop(0, NCHUNK // 2, pair_body, 0)


def kernel(x, edge_index):
    xt = x.T.reshape(-1)
    out = _ef_kernel(xt, edge_index[0], edge_index[1])
    return out.astype(jnp.bool_)
